# baseline (speedup 1.0000x reference)
import functools

import jax
import jax.numpy as jnp
from jax import lax
from jax.experimental import pallas as pl
from jax.experimental.pallas import tpu as pltpu
from jax.experimental.pallas import tpu_sc as plsc

_B = 256
_F = 4 * 128 * 128
_BB = 8


def _make_coeff_gather():
    info = plsc.get_sparse_core_info()
    nc, ns = info.num_cores, info.num_subcores
    nw = nc * ns
    bpw = _B // nw

    mesh = plsc.VectorSubcoreMesh(core_axis_name="c", subcore_axis_name="s")

    @functools.partial(
        pl.kernel,
        mesh=mesh,
        out_type=(
            jax.ShapeDtypeStruct((_B,), jnp.float32),
            jax.ShapeDtypeStruct((_B,), jnp.float32),
        ),
        scratch_types=[
            pltpu.VMEM((bpw,), jnp.int32),
            pltpu.VMEM((bpw,), jnp.float32),
            pltpu.VMEM((bpw,), jnp.float32),
            pltpu.SemaphoreType.DMA,
            pltpu.SemaphoreType.DMA,
        ],
    )
    def gather(t_hbm, sac_hbm, somac_hbm, a_out, b_out, idx_v, a_v, b_v,
               sem_a, sem_b):
        wid = lax.axis_index("s") * nc + lax.axis_index("c")
        base = wid * bpw
        pltpu.sync_copy(t_hbm.at[pl.ds(base, bpw)], idx_v)
        ca = pltpu.async_copy(sac_hbm.at[idx_v], a_v, sem_a)
        cb = pltpu.async_copy(somac_hbm.at[idx_v], b_v, sem_b)
        ca.wait()
        cb.wait()
        pltpu.sync_copy(a_v, a_out.at[pl.ds(base, bpw)])
        pltpu.sync_copy(b_v, b_out.at[pl.ds(base, bpw)])

    return gather


_coeff_gather = _make_coeff_gather()


def _blend_body(a_ref, b_ref, x_ref, n_ref, o_ref):
    o_ref[...] = a_ref[...] * x_ref[...] + b_ref[...] * n_ref[...]


@jax.jit
def kernel(x_start, noise, t, sqrt_alphas_cumprod, sqrt_one_minus_alphas_cumprod):
    x2 = x_start.reshape(_B, _F)
    n2 = noise.reshape(_B, _F)
    a_vec, b_vec = _coeff_gather(
        t.astype(jnp.int32), sqrt_alphas_cumprod, sqrt_one_minus_alphas_cumprod)
    a2 = a_vec.reshape(_B, 1)
    b2 = b_vec.reshape(_B, 1)
    out = pl.pallas_call(
        _blend_body,
        grid=(_B // _BB,),
        in_specs=[
            pl.BlockSpec((_BB, 1), lambda i: (i, 0)),
            pl.BlockSpec((_BB, 1), lambda i: (i, 0)),
            pl.BlockSpec((_BB, _F), lambda i: (i, 0)),
            pl.BlockSpec((_BB, _F), lambda i: (i, 0)),
        ],
        out_specs=pl.BlockSpec((_BB, _F), lambda i: (i, 0)),
        out_shape=jax.ShapeDtypeStruct((_B, _F), jnp.float32),
    )(a2, b2, x2, n2)
    return out.reshape(x_start.shape)


# speedup vs baseline: 3.1064x; 3.1064x over previous
import functools

import jax
import jax.numpy as jnp
from jax import lax
from jax.experimental import pallas as pl
from jax.experimental.pallas import tpu as pltpu
from jax.experimental.pallas import tpu_sc as plsc

_B = 256
_F = 4 * 128 * 128
_BB = 8


def _make_coeff_gather():
    info = plsc.get_sparse_core_info()
    nc, ns = info.num_cores, info.num_subcores
    nw = nc * ns
    bpw = _B // nw

    mesh = plsc.VectorSubcoreMesh(core_axis_name="c", subcore_axis_name="s")

    @functools.partial(
        pl.kernel,
        mesh=mesh,
        out_type=(
            jax.ShapeDtypeStruct((_B,), jnp.float32),
            jax.ShapeDtypeStruct((_B,), jnp.float32),
        ),
        scratch_types=[
            pltpu.VMEM((bpw,), jnp.int32),
            pltpu.VMEM((bpw,), jnp.float32),
            pltpu.VMEM((bpw,), jnp.float32),
            pltpu.SemaphoreType.DMA,
            pltpu.SemaphoreType.DMA,
        ],
    )
    def gather(t_hbm, sac_hbm, somac_hbm, a_out, b_out, idx_v, a_v, b_v,
               sem_a, sem_b):
        wid = lax.axis_index("s") * nc + lax.axis_index("c")
        base = wid * bpw
        pltpu.sync_copy(t_hbm.at[pl.ds(base, bpw)], idx_v)
        ca = pltpu.async_copy(sac_hbm.at[idx_v], a_v, sem_a)
        cb = pltpu.async_copy(somac_hbm.at[idx_v], b_v, sem_b)
        ca.wait()
        cb.wait()
        pltpu.sync_copy(a_v, a_out.at[pl.ds(base, bpw)])
        pltpu.sync_copy(b_v, b_out.at[pl.ds(base, bpw)])

    return gather


_coeff_gather = _make_coeff_gather()


def _blend_body(a_ref, b_ref, x_ref, n_ref, o_ref):
    base = pl.program_id(0) * _BB
    for j in range(_BB):
        o_ref[j] = a_ref[base + j] * x_ref[j] + b_ref[base + j] * n_ref[j]


@jax.jit
def kernel(x_start, noise, t, sqrt_alphas_cumprod, sqrt_one_minus_alphas_cumprod):
    a_vec, b_vec = _coeff_gather(
        t.astype(jnp.int32), sqrt_alphas_cumprod, sqrt_one_minus_alphas_cumprod)
    c, h, w = x_start.shape[1:]
    blk = (_BB, c, h, w)
    dense_spec = pl.BlockSpec(blk, lambda i: (i, 0, 0, 0))
    return pl.pallas_call(
        _blend_body,
        grid=(_B // _BB,),
        in_specs=[
            pl.BlockSpec(memory_space=pltpu.SMEM),
            pl.BlockSpec(memory_space=pltpu.SMEM),
            dense_spec,
            dense_spec,
        ],
        out_specs=dense_spec,
        out_shape=jax.ShapeDtypeStruct(x_start.shape, jnp.float32),
    )(a_vec, b_vec, x_start, noise)


# speedup vs baseline: 3.1654x; 1.0190x over previous
import functools

import jax
import jax.numpy as jnp
from jax import lax
from jax.experimental import pallas as pl
from jax.experimental.pallas import tpu as pltpu
from jax.experimental.pallas import tpu_sc as plsc

_B = 256
_F = 4 * 128 * 128
_BB = 16


def _make_coeff_gather():
    info = plsc.get_sparse_core_info()
    nc, ns = info.num_cores, info.num_subcores
    nw = nc * ns
    bpw = _B // nw

    mesh = plsc.VectorSubcoreMesh(core_axis_name="c", subcore_axis_name="s")

    @functools.partial(
        pl.kernel,
        mesh=mesh,
        out_type=(
            jax.ShapeDtypeStruct((_B,), jnp.float32),
            jax.ShapeDtypeStruct((_B,), jnp.float32),
        ),
        scratch_types=[
            pltpu.VMEM((bpw,), jnp.int32),
            pltpu.VMEM((bpw,), jnp.float32),
            pltpu.VMEM((bpw,), jnp.float32),
            pltpu.SemaphoreType.DMA,
            pltpu.SemaphoreType.DMA,
        ],
    )
    def gather(t_hbm, sac_hbm, somac_hbm, a_out, b_out, idx_v, a_v, b_v,
               sem_a, sem_b):
        wid = lax.axis_index("s") * nc + lax.axis_index("c")
        base = wid * bpw
        pltpu.sync_copy(t_hbm.at[pl.ds(base, bpw)], idx_v)
        ca = pltpu.async_copy(sac_hbm.at[idx_v], a_v, sem_a)
        cb = pltpu.async_copy(somac_hbm.at[idx_v], b_v, sem_b)
        ca.wait()
        cb.wait()
        pltpu.sync_copy(a_v, a_out.at[pl.ds(base, bpw)])
        pltpu.sync_copy(b_v, b_out.at[pl.ds(base, bpw)])

    return gather


_coeff_gather = _make_coeff_gather()


def _blend_body(a_ref, b_ref, x_ref, n_ref, o_ref):
    base = pl.program_id(0) * _BB
    for j in range(_BB):
        o_ref[j] = a_ref[base + j] * x_ref[j] + b_ref[base + j] * n_ref[j]


@jax.jit
def kernel(x_start, noise, t, sqrt_alphas_cumprod, sqrt_one_minus_alphas_cumprod):
    a_vec, b_vec = _coeff_gather(
        t.astype(jnp.int32), sqrt_alphas_cumprod, sqrt_one_minus_alphas_cumprod)
    c, h, w = x_start.shape[1:]
    blk = (_BB, c, h, w)
    dense_spec = pl.BlockSpec(blk, lambda i: (i, 0, 0, 0))
    return pl.pallas_call(
        _blend_body,
        grid=(_B // _BB,),
        in_specs=[
            pl.BlockSpec(memory_space=pltpu.SMEM),
            pl.BlockSpec(memory_space=pltpu.SMEM),
            dense_spec,
            dense_spec,
        ],
        out_specs=dense_spec,
        out_shape=jax.ShapeDtypeStruct(x_start.shape, jnp.float32),
    )(a_vec, b_vec, x_start, noise)
